# baseline (speedup 1.0000x reference)
import functools

import jax
import jax.numpy as jnp
from jax import lax
from jax.experimental import pallas as pl
from jax.experimental.pallas import tpu as pltpu
from jax.experimental.pallas import tpu_sc as plsc

N_TOK = 4096
C_DIM = 1024
H_DIM = 4096
N_EXP = 8
TOPK = 2
NK = N_TOK * TOPK
TILE = 128
M_MAX = NK // TILE + N_EXP
P_MAX = M_MAX * TILE
LANES = 16
DW = 32
CW = 32


def _routing_body(x_ref, gw_ref, dst_ref, te_ref, wb0_ref, wb1_ref):
    x = x_ref[...]
    gw = gw_ref[...]
    s = jnp.dot(x, gw, preferred_element_type=jnp.float32)
    ids = lax.broadcasted_iota(jnp.int32, s.shape, 1)
    m1 = jnp.max(s, axis=1, keepdims=True)
    i1 = jnp.min(jnp.where(s == m1, ids, N_EXP), axis=1, keepdims=True)
    sm = jnp.where(ids == i1, -jnp.inf, s)
    m2 = jnp.max(sm, axis=1, keepdims=True)
    i2 = jnp.min(jnp.where(sm == m2, ids, N_EXP), axis=1, keepdims=True)
    e2 = jnp.exp(m2 - m1)
    w1 = 1.0 / (1.0 + e2)
    w2 = e2 / (1.0 + e2)

    e_all = jnp.concatenate([i1, i2], axis=0)
    oh = (e_all == lax.broadcasted_iota(jnp.int32, (NK, N_EXP), 1))
    oh = oh.astype(jnp.int32)
    c = oh
    sh = 1
    while sh < NK:
        c = c + jnp.concatenate(
            [jnp.zeros((sh, N_EXP), jnp.int32), c[:-sh, :]], axis=0)
        sh *= 2
    counts = c[NK - 1:NK, :]
    pc = ((counts + TILE - 1) // TILE) * TILE
    t = pc
    for lsh in (1, 2, 4):
        t = t + jnp.concatenate(
            [jnp.zeros((1, lsh), jnp.int32), t[:, :-lsh]], axis=1)
    pad_excl = t - pc
    rank = jnp.sum(c * oh, axis=1, keepdims=True) - 1
    base = jnp.sum(pad_excl * oh, axis=1, keepdims=True)
    dst_ref[...] = base + rank
    mt = lax.broadcasted_iota(jnp.int32, (1, 128), 1) * TILE
    te = jnp.zeros((1, 128), jnp.int32)
    for e in range(1, N_EXP):
        te = te + (pad_excl[:, e:e + 1] <= mt).astype(jnp.int32)
    te_ref[...] = te
    wb0_ref[...] = jnp.broadcast_to(w1, (N_TOK, LANES))
    wb1_ref[...] = jnp.broadcast_to(w2, (N_TOK, LANES))


def _routing(x_flat, gate_w):
    return pl.pallas_call(
        _routing_body,
        out_shape=[
            jax.ShapeDtypeStruct((NK, 1), jnp.int32),
            jax.ShapeDtypeStruct((1, 128), jnp.int32),
            jax.ShapeDtypeStruct((N_TOK, LANES), jnp.float32),
            jax.ShapeDtypeStruct((N_TOK, LANES), jnp.float32),
        ],
    )(x_flat, gate_w)


NW = 32
TPW = N_TOK // NW


def _cast_body(x_ref, o_ref):
    o_ref[...] = x_ref[...].astype(jnp.bfloat16)


def _cast(x_flat):
    return pl.pallas_call(
        _cast_body,
        grid=(8,),
        in_specs=[pl.BlockSpec((N_TOK // 8, C_DIM), lambda i: (i, 0))],
        out_specs=pl.BlockSpec((N_TOK // 8, C_DIM), lambda i: (i, 0)),
        out_shape=jax.ShapeDtypeStruct((N_TOK, C_DIM), jnp.bfloat16),
    )(x_flat)


def _dispatch(x_i32, i0r, i1r):
    mesh = plsc.VectorSubcoreMesh(core_axis_name="core",
                                  subcore_axis_name="subcore")

    @functools.partial(
        pl.kernel,
        out_type=jax.ShapeDtypeStruct((P_MAX, C_DIM // 2), jnp.int32),
        mesh=mesh,
        scratch_types=[pltpu.VMEM((TPW,), jnp.int32),
                       pltpu.VMEM((TPW,), jnp.int32),
                       pltpu.VMEM((TPW, C_DIM // 2), jnp.int32),
                       pltpu.SemaphoreType.DMA,
                       pltpu.SemaphoreType.DMA])
    def k(x_hbm, i0_hbm, i1_hbm, xs_hbm, idx0_v, idx1_v, xbuf, s0, s1):
        w = lax.axis_index("core") * 16 + lax.axis_index("subcore")
        pltpu.sync_copy(i0_hbm.at[w], idx0_v)
        pltpu.sync_copy(i1_hbm.at[w], idx1_v)
        pltpu.sync_copy(x_hbm.at[pl.ds(w * TPW, TPW)], xbuf)
        c0 = pltpu.async_copy(xbuf, xs_hbm.at[idx0_v], s0)
        c1 = pltpu.async_copy(xbuf, xs_hbm.at[idx1_v], s1)
        c0.wait()
        c1.wait()

    return k(x_i32, i0r, i1r)


def _mlp_body(te_ref, x_ref, wfc_ref, bfc_ref, wpj_ref, bpj_ref, o_ref):
    del te_ref
    xb = x_ref[...]
    h = lax.dot_general(xb, wfc_ref[0], (((1,), (0,)), ((), ())),
                        preferred_element_type=jnp.float32)
    h = h + bfc_ref[0]
    h = 0.5 * h * (1.0 + lax.erf(h * 0.7071067811865476))
    hb = h.astype(jnp.bfloat16)
    o = lax.dot_general(hb, wpj_ref[0], (((1,), (0,)), ((), ())),
                        preferred_element_type=jnp.float32)
    o_ref[...] = o + bpj_ref[0]


def _mlp(te, x_sorted, w_fc, b_fc, w_proj, b_proj):
    grid_spec = pltpu.PrefetchScalarGridSpec(
        num_scalar_prefetch=1,
        grid=(M_MAX,),
        in_specs=[
            pl.BlockSpec((TILE, C_DIM), lambda m, te: (m, 0)),
            pl.BlockSpec((1, C_DIM, H_DIM), lambda m, te: (te[m], 0, 0)),
            pl.BlockSpec((1, 1, H_DIM), lambda m, te: (te[m], 0, 0)),
            pl.BlockSpec((1, H_DIM, C_DIM), lambda m, te: (te[m], 0, 0)),
            pl.BlockSpec((1, 1, C_DIM), lambda m, te: (te[m], 0, 0)),
        ],
        out_specs=pl.BlockSpec((TILE, C_DIM), lambda m, te: (m, 0)),
    )
    return pl.pallas_call(
        _mlp_body,
        grid_spec=grid_spec,
        out_shape=jax.ShapeDtypeStruct((P_MAX, C_DIM), jnp.float32),
    )(te, x_sorted, w_fc, b_fc, w_proj, b_proj)


def _combine(out_sorted, i0r, i1r, wb0, wb1):
    mesh = plsc.VectorSubcoreMesh(core_axis_name="core",
                                  subcore_axis_name="subcore")

    @functools.partial(
        pl.kernel,
        out_type=jax.ShapeDtypeStruct((N_TOK, C_DIM), jnp.float32),
        mesh=mesh,
        scratch_types=[pltpu.VMEM((TPW // CW, CW), jnp.int32),
                       pltpu.VMEM((TPW // CW, CW), jnp.int32),
                       pltpu.VMEM((CW, LANES), jnp.float32),
                       pltpu.VMEM((CW, LANES), jnp.float32),
                       pltpu.VMEM((CW, C_DIM), jnp.float32),
                       pltpu.VMEM((CW, C_DIM), jnp.float32),
                       pltpu.VMEM((CW, C_DIM), jnp.float32),
                       pltpu.SemaphoreType.DMA,
                       pltpu.SemaphoreType.DMA])
    def k(os_hbm, i0_hbm, i1_hbm, w0_hbm, w1_hbm, y_hbm,
          idx0_v, idx1_v, wb0_v, wb1_v, ra, rb, ybuf, s0, s1):
        w = lax.axis_index("core") * 16 + lax.axis_index("subcore")
        pltpu.sync_copy(i0_hbm.at[w], idx0_v)
        pltpu.sync_copy(i1_hbm.at[w], idx1_v)

        @pl.loop(0, TPW // CW)
        def _(cc):
            c0 = pltpu.async_copy(os_hbm.at[idx0_v.at[cc]], ra, s0)
            c1 = pltpu.async_copy(os_hbm.at[idx1_v.at[cc]], rb, s1)
            pltpu.sync_copy(w0_hbm.at[pl.ds(w * TPW + cc * CW, CW)], wb0_v)
            pltpu.sync_copy(w1_hbm.at[pl.ds(w * TPW + cc * CW, CW)], wb1_v)
            c0.wait()
            c1.wait()

            @pl.loop(0, CW)
            def _(i):
                wa = wb0_v[i, :]
                wb = wb1_v[i, :]
                for ch in range(C_DIM // LANES):
                    sl = pl.ds(ch * LANES, LANES)
                    ybuf[i, sl] = wa * ra[i, sl] + wb * rb[i, sl]

            pltpu.sync_copy(ybuf, y_hbm.at[pl.ds(w * TPW + cc * CW, CW)])

    return k(out_sorted, i0r.reshape(NW, TPW // CW, CW),
             i1r.reshape(NW, TPW // CW, CW), wb0, wb1)


def kernel(x, gate_w, w_fc, b_fc, w_proj, b_proj):
    bx, tx, c = x.shape
    x_flat = x.reshape(-1, c)
    dst, te, wb0, wb1 = _routing(x_flat, gate_w)
    dstr = dst.reshape(TOPK, NW, TPW)
    i0 = dstr[0]
    i1 = dstr[1]
    te_arr = te[0, :M_MAX]
    x_i32 = lax.bitcast_convert_type(
        _cast(x_flat).reshape(N_TOK, C_DIM // 2, 2), jnp.int32)
    xs_i32 = _dispatch(x_i32, i0, i1)
    x_sorted = lax.bitcast_convert_type(
        xs_i32, jnp.bfloat16).reshape(P_MAX, C_DIM)
    out_sorted = _mlp(
        te_arr, x_sorted,
        w_fc.astype(jnp.bfloat16), b_fc.reshape(N_EXP, 1, H_DIM),
        w_proj.astype(jnp.bfloat16), b_proj.reshape(N_EXP, 1, C_DIM))
    y = _combine(out_sorted, i0, i1, wb0, wb1)
    return y.reshape(bx, tx, c), jnp.asarray(0.0, x.dtype)


# speedup vs baseline: 1.5916x; 1.5916x over previous
import functools

import jax
import jax.numpy as jnp
from jax import lax
from jax.experimental import pallas as pl
from jax.experimental.pallas import tpu as pltpu
from jax.experimental.pallas import tpu_sc as plsc

N_TOK = 4096
C_DIM = 1024
H_DIM = 4096
N_EXP = 8
TOPK = 2
NK = N_TOK * TOPK
TILE = 128
M_MAX = NK // TILE + N_EXP
P_MAX = M_MAX * TILE
LANES = 16
DW = 32
CW = 32


def _routing_body(x_ref, gw_ref, dst_ref, te_ref, wb0_ref, wb1_ref):
    x = x_ref[...]
    gw = gw_ref[...]
    s = jnp.dot(x, gw, preferred_element_type=jnp.float32)
    ids = lax.broadcasted_iota(jnp.int32, s.shape, 1)
    m1 = jnp.max(s, axis=1, keepdims=True)
    i1 = jnp.min(jnp.where(s == m1, ids, N_EXP), axis=1, keepdims=True)
    sm = jnp.where(ids == i1, -jnp.inf, s)
    m2 = jnp.max(sm, axis=1, keepdims=True)
    i2 = jnp.min(jnp.where(sm == m2, ids, N_EXP), axis=1, keepdims=True)
    e2 = jnp.exp(m2 - m1)
    w1 = 1.0 / (1.0 + e2)
    w2 = e2 / (1.0 + e2)

    e_all = jnp.concatenate([i1, i2], axis=0)
    oh = (e_all == lax.broadcasted_iota(jnp.int32, (NK, N_EXP), 1))
    oh = oh.astype(jnp.int32)
    c = oh
    sh = 1
    while sh < NK:
        c = c + jnp.concatenate(
            [jnp.zeros((sh, N_EXP), jnp.int32), c[:-sh, :]], axis=0)
        sh *= 2
    counts = c[NK - 1:NK, :]
    pc = ((counts + TILE - 1) // TILE) * TILE
    t = pc
    for lsh in (1, 2, 4):
        t = t + jnp.concatenate(
            [jnp.zeros((1, lsh), jnp.int32), t[:, :-lsh]], axis=1)
    pad_excl = t - pc
    rank = jnp.sum(c * oh, axis=1, keepdims=True) - 1
    base = jnp.sum(pad_excl * oh, axis=1, keepdims=True)
    dst_ref[...] = base + rank
    mt = lax.broadcasted_iota(jnp.int32, (1, 128), 1) * TILE
    te = jnp.zeros((1, 128), jnp.int32)
    for e in range(1, N_EXP):
        te = te + (pad_excl[:, e:e + 1] <= mt).astype(jnp.int32)
    te_ref[...] = te
    wb0_ref[...] = jnp.broadcast_to(w1, (N_TOK, LANES))
    wb1_ref[...] = jnp.broadcast_to(w2, (N_TOK, LANES))


def _routing(x_flat, gate_w):
    return pl.pallas_call(
        _routing_body,
        out_shape=[
            jax.ShapeDtypeStruct((NK, 1), jnp.int32),
            jax.ShapeDtypeStruct((1, 128), jnp.int32),
            jax.ShapeDtypeStruct((N_TOK, LANES), jnp.float32),
            jax.ShapeDtypeStruct((N_TOK, LANES), jnp.float32),
        ],
    )(x_flat, gate_w)


NW = 32
TPW = N_TOK // NW


def _rtne_bf16_bits(xu):
    return (xu + 0x7FFF + ((xu >> 16) & 1)) >> 16


def _cast_body(x_ref, o_ref):
    x = x_ref[...]
    xu = lax.bitcast_convert_type(x, jnp.uint32)
    lo = _rtne_bf16_bits(xu[:, :C_DIM // 2])
    hi = _rtne_bf16_bits(xu[:, C_DIM // 2:])
    o_ref[...] = lax.bitcast_convert_type(lo | (hi << 16), jnp.int32)


def _cast(x_flat):
    return pl.pallas_call(
        _cast_body,
        grid=(8,),
        in_specs=[pl.BlockSpec((N_TOK // 8, C_DIM), lambda i: (i, 0))],
        out_specs=pl.BlockSpec((N_TOK // 8, C_DIM // 2), lambda i: (i, 0)),
        out_shape=jax.ShapeDtypeStruct((N_TOK, C_DIM // 2), jnp.int32),
    )(x_flat)


def _dispatch(x_i32, i0r, i1r):
    mesh = plsc.VectorSubcoreMesh(core_axis_name="core",
                                  subcore_axis_name="subcore")

    @functools.partial(
        pl.kernel,
        out_type=jax.ShapeDtypeStruct((P_MAX, C_DIM // 2), jnp.int32),
        mesh=mesh,
        scratch_types=[pltpu.VMEM((TPW,), jnp.int32),
                       pltpu.VMEM((TPW,), jnp.int32),
                       pltpu.VMEM((TPW, C_DIM // 2), jnp.int32),
                       pltpu.SemaphoreType.DMA,
                       pltpu.SemaphoreType.DMA])
    def k(x_hbm, i0_hbm, i1_hbm, xs_hbm, idx0_v, idx1_v, xbuf, s0, s1):
        w = lax.axis_index("core") * 16 + lax.axis_index("subcore")
        pltpu.sync_copy(i0_hbm.at[w], idx0_v)
        pltpu.sync_copy(i1_hbm.at[w], idx1_v)
        pltpu.sync_copy(x_hbm.at[pl.ds(w * TPW, TPW)], xbuf)
        c0 = pltpu.async_copy(xbuf, xs_hbm.at[idx0_v], s0)
        c1 = pltpu.async_copy(xbuf, xs_hbm.at[idx1_v], s1)
        c0.wait()
        c1.wait()

    return k(x_i32, i0r, i1r)


def _mlp_body(te_ref, x_ref, wfc_ref, bfc_ref, wpj_ref, bpj_ref, o_ref):
    del te_ref
    wu = lax.bitcast_convert_type(x_ref[...], jnp.uint32)
    xlo = lax.bitcast_convert_type(wu << 16, jnp.float32).astype(jnp.bfloat16)
    xhi = lax.bitcast_convert_type(wu & jnp.uint32(0xFFFF0000),
                                   jnp.float32).astype(jnp.bfloat16)
    h = (lax.dot_general(xlo, wfc_ref[0, :C_DIM // 2],
                         (((1,), (0,)), ((), ())),
                         preferred_element_type=jnp.float32)
         + lax.dot_general(xhi, wfc_ref[0, C_DIM // 2:],
                           (((1,), (0,)), ((), ())),
                           preferred_element_type=jnp.float32))
    h = h + bfc_ref[0]
    h = 0.5 * h * (1.0 + lax.erf(h * 0.7071067811865476))
    hb = h.astype(jnp.bfloat16)
    o = lax.dot_general(hb, wpj_ref[0], (((1,), (0,)), ((), ())),
                        preferred_element_type=jnp.float32)
    o_ref[...] = o + bpj_ref[0]


def _mlp(te, x_sorted, w_fc, b_fc, w_proj, b_proj):
    grid_spec = pltpu.PrefetchScalarGridSpec(
        num_scalar_prefetch=1,
        grid=(M_MAX,),
        in_specs=[
            pl.BlockSpec((TILE, C_DIM // 2), lambda m, te: (m, 0)),
            pl.BlockSpec((1, C_DIM, H_DIM), lambda m, te: (te[m], 0, 0)),
            pl.BlockSpec((1, 1, H_DIM), lambda m, te: (te[m], 0, 0)),
            pl.BlockSpec((1, H_DIM, C_DIM), lambda m, te: (te[m], 0, 0)),
            pl.BlockSpec((1, 1, C_DIM), lambda m, te: (te[m], 0, 0)),
        ],
        out_specs=pl.BlockSpec((TILE, C_DIM), lambda m, te: (m, 0)),
    )
    return pl.pallas_call(
        _mlp_body,
        grid_spec=grid_spec,
        out_shape=jax.ShapeDtypeStruct((P_MAX, C_DIM), jnp.float32),
    )(te, x_sorted, w_fc, b_fc, w_proj, b_proj)


def _combine(out_sorted, i0r, i1r, wb0, wb1):
    mesh = plsc.VectorSubcoreMesh(core_axis_name="core",
                                  subcore_axis_name="subcore")

    @functools.partial(
        pl.kernel,
        out_type=jax.ShapeDtypeStruct((N_TOK, C_DIM), jnp.float32),
        mesh=mesh,
        scratch_types=[pltpu.VMEM((TPW // CW, CW), jnp.int32),
                       pltpu.VMEM((TPW // CW, CW), jnp.int32),
                       pltpu.VMEM((CW, LANES), jnp.float32),
                       pltpu.VMEM((CW, LANES), jnp.float32),
                       pltpu.VMEM((CW, C_DIM), jnp.float32),
                       pltpu.VMEM((CW, C_DIM), jnp.float32),
                       pltpu.VMEM((CW, C_DIM), jnp.float32),
                       pltpu.SemaphoreType.DMA,
                       pltpu.SemaphoreType.DMA])
    def k(os_hbm, i0_hbm, i1_hbm, w0_hbm, w1_hbm, y_hbm,
          idx0_v, idx1_v, wb0_v, wb1_v, ra, rb, ybuf, s0, s1):
        w = lax.axis_index("core") * 16 + lax.axis_index("subcore")
        pltpu.sync_copy(i0_hbm.at[w], idx0_v)
        pltpu.sync_copy(i1_hbm.at[w], idx1_v)

        @pl.loop(0, TPW // CW)
        def _(cc):
            c0 = pltpu.async_copy(os_hbm.at[idx0_v.at[cc]], ra, s0)
            c1 = pltpu.async_copy(os_hbm.at[idx1_v.at[cc]], rb, s1)
            pltpu.sync_copy(w0_hbm.at[pl.ds(w * TPW + cc * CW, CW)], wb0_v)
            pltpu.sync_copy(w1_hbm.at[pl.ds(w * TPW + cc * CW, CW)], wb1_v)
            c0.wait()
            c1.wait()

            @pl.loop(0, CW)
            def _(i):
                wa = wb0_v[i, :]
                wb = wb1_v[i, :]
                for ch in range(C_DIM // LANES):
                    sl = pl.ds(ch * LANES, LANES)
                    ybuf[i, sl] = wa * ra[i, sl] + wb * rb[i, sl]

            pltpu.sync_copy(ybuf, y_hbm.at[pl.ds(w * TPW + cc * CW, CW)])

    return k(out_sorted, i0r.reshape(NW, TPW // CW, CW),
             i1r.reshape(NW, TPW // CW, CW), wb0, wb1)


def kernel(x, gate_w, w_fc, b_fc, w_proj, b_proj):
    bx, tx, c = x.shape
    x_flat = x.reshape(-1, c)
    dst, te, wb0, wb1 = _routing(x_flat, gate_w)
    dstr = dst.reshape(TOPK, NW, TPW)
    i0 = dstr[0]
    i1 = dstr[1]
    te_arr = te[0, :M_MAX]
    x_sorted = _dispatch(_cast(x_flat), i0, i1)
    out_sorted = _mlp(
        te_arr, x_sorted,
        w_fc.astype(jnp.bfloat16), b_fc.reshape(N_EXP, 1, H_DIM),
        w_proj.astype(jnp.bfloat16), b_proj.reshape(N_EXP, 1, C_DIM))
    y = _combine(out_sorted, i0, i1, wb0, wb1)
    return y.reshape(bx, tx, c), jnp.asarray(0.0, x.dtype)


# speedup vs baseline: 1.6284x; 1.0231x over previous
import functools

import jax
import jax.numpy as jnp
from jax import lax
from jax.experimental import pallas as pl
from jax.experimental.pallas import tpu as pltpu
from jax.experimental.pallas import tpu_sc as plsc

N_TOK = 4096
C_DIM = 1024
H_DIM = 4096
N_EXP = 8
TOPK = 2
NK = N_TOK * TOPK
TILE = 256
M_MAX = NK // TILE + N_EXP
P_MAX = M_MAX * TILE
LANES = 16
DW = 32
CW = 32


def _routing_body(x_ref, gw_ref, dst_ref, te_ref, wb0_ref, wb1_ref):
    x = x_ref[...]
    gw = gw_ref[...]
    s = jnp.dot(x, gw, preferred_element_type=jnp.float32)
    ids = lax.broadcasted_iota(jnp.int32, s.shape, 1)
    m1 = jnp.max(s, axis=1, keepdims=True)
    i1 = jnp.min(jnp.where(s == m1, ids, N_EXP), axis=1, keepdims=True)
    sm = jnp.where(ids == i1, -jnp.inf, s)
    m2 = jnp.max(sm, axis=1, keepdims=True)
    i2 = jnp.min(jnp.where(sm == m2, ids, N_EXP), axis=1, keepdims=True)
    e2 = jnp.exp(m2 - m1)
    w1 = 1.0 / (1.0 + e2)
    w2 = e2 / (1.0 + e2)

    e_all = jnp.concatenate([i1, i2], axis=0)
    oh = (e_all == lax.broadcasted_iota(jnp.int32, (NK, N_EXP), 1))
    oh = oh.astype(jnp.int32)
    c = oh
    sh = 1
    while sh < NK:
        c = c + jnp.concatenate(
            [jnp.zeros((sh, N_EXP), jnp.int32), c[:-sh, :]], axis=0)
        sh *= 2
    counts = c[NK - 1:NK, :]
    pc = ((counts + TILE - 1) // TILE) * TILE
    t = pc
    for lsh in (1, 2, 4):
        t = t + jnp.concatenate(
            [jnp.zeros((1, lsh), jnp.int32), t[:, :-lsh]], axis=1)
    pad_excl = t - pc
    rank = jnp.sum(c * oh, axis=1, keepdims=True) - 1
    base = jnp.sum(pad_excl * oh, axis=1, keepdims=True)
    dst_ref[...] = base + rank
    mt = lax.broadcasted_iota(jnp.int32, (1, 128), 1) * TILE
    te = jnp.zeros((1, 128), jnp.int32)
    for e in range(1, N_EXP):
        te = te + (pad_excl[:, e:e + 1] <= mt).astype(jnp.int32)
    te_ref[...] = te
    wb0_ref[...] = jnp.broadcast_to(w1, (N_TOK, LANES))
    wb1_ref[...] = jnp.broadcast_to(w2, (N_TOK, LANES))


def _routing(x_flat, gate_w):
    return pl.pallas_call(
        _routing_body,
        out_shape=[
            jax.ShapeDtypeStruct((NK, 1), jnp.int32),
            jax.ShapeDtypeStruct((1, 128), jnp.int32),
            jax.ShapeDtypeStruct((N_TOK, LANES), jnp.float32),
            jax.ShapeDtypeStruct((N_TOK, LANES), jnp.float32),
        ],
    )(x_flat, gate_w)


NW = 32
TPW = N_TOK // NW


def _rtne_bf16_bits(xu):
    return (xu + 0x7FFF + ((xu >> 16) & 1)) >> 16


def _cast_body(x_ref, o_ref):
    x = x_ref[...]
    xu = lax.bitcast_convert_type(x, jnp.uint32)
    lo = _rtne_bf16_bits(xu[:, :C_DIM // 2])
    hi = _rtne_bf16_bits(xu[:, C_DIM // 2:])
    o_ref[...] = lax.bitcast_convert_type(lo | (hi << 16), jnp.int32)


def _cast(x_flat):
    return pl.pallas_call(
        _cast_body,
        grid=(8,),
        in_specs=[pl.BlockSpec((N_TOK // 8, C_DIM), lambda i: (i, 0))],
        out_specs=pl.BlockSpec((N_TOK // 8, C_DIM // 2), lambda i: (i, 0)),
        out_shape=jax.ShapeDtypeStruct((N_TOK, C_DIM // 2), jnp.int32),
    )(x_flat)


def _dispatch(x_i32, i0r, i1r):
    mesh = plsc.VectorSubcoreMesh(core_axis_name="core",
                                  subcore_axis_name="subcore")

    @functools.partial(
        pl.kernel,
        out_type=jax.ShapeDtypeStruct((P_MAX, C_DIM // 2), jnp.int32),
        mesh=mesh,
        scratch_types=[pltpu.VMEM((TPW,), jnp.int32),
                       pltpu.VMEM((TPW,), jnp.int32),
                       pltpu.VMEM((TPW, C_DIM // 2), jnp.int32),
                       pltpu.SemaphoreType.DMA,
                       pltpu.SemaphoreType.DMA])
    def k(x_hbm, i0_hbm, i1_hbm, xs_hbm, idx0_v, idx1_v, xbuf, s0, s1):
        w = lax.axis_index("core") * 16 + lax.axis_index("subcore")
        pltpu.sync_copy(i0_hbm.at[w], idx0_v)
        pltpu.sync_copy(i1_hbm.at[w], idx1_v)
        pltpu.sync_copy(x_hbm.at[pl.ds(w * TPW, TPW)], xbuf)
        c0 = pltpu.async_copy(xbuf, xs_hbm.at[idx0_v], s0)
        c1 = pltpu.async_copy(xbuf, xs_hbm.at[idx1_v], s1)
        c0.wait()
        c1.wait()

    return k(x_i32, i0r, i1r)


def _mlp_body(te_ref, x_ref, wfc_ref, bfc_ref, wpj_ref, bpj_ref, o_ref):
    del te_ref
    wu = lax.bitcast_convert_type(x_ref[...], jnp.uint32)
    xlo = lax.bitcast_convert_type(wu << 16, jnp.float32).astype(jnp.bfloat16)
    xhi = lax.bitcast_convert_type(wu & jnp.uint32(0xFFFF0000),
                                   jnp.float32).astype(jnp.bfloat16)
    h = (lax.dot_general(xlo, wfc_ref[0, :C_DIM // 2],
                         (((1,), (0,)), ((), ())),
                         preferred_element_type=jnp.float32)
         + lax.dot_general(xhi, wfc_ref[0, C_DIM // 2:],
                           (((1,), (0,)), ((), ())),
                           preferred_element_type=jnp.float32))
    h = h + bfc_ref[0]
    h = 0.5 * h * (1.0 + lax.erf(h * 0.7071067811865476))
    hb = h.astype(jnp.bfloat16)
    o = lax.dot_general(hb, wpj_ref[0], (((1,), (0,)), ((), ())),
                        preferred_element_type=jnp.float32)
    o_ref[...] = o + bpj_ref[0]


def _mlp(te, x_sorted, w_fc, b_fc, w_proj, b_proj):
    grid_spec = pltpu.PrefetchScalarGridSpec(
        num_scalar_prefetch=1,
        grid=(M_MAX,),
        in_specs=[
            pl.BlockSpec((TILE, C_DIM // 2), lambda m, te: (m, 0)),
            pl.BlockSpec((1, C_DIM, H_DIM), lambda m, te: (te[m], 0, 0)),
            pl.BlockSpec((1, 1, H_DIM), lambda m, te: (te[m], 0, 0)),
            pl.BlockSpec((1, H_DIM, C_DIM), lambda m, te: (te[m], 0, 0)),
            pl.BlockSpec((1, 1, C_DIM), lambda m, te: (te[m], 0, 0)),
        ],
        out_specs=pl.BlockSpec((TILE, C_DIM), lambda m, te: (m, 0)),
    )
    return pl.pallas_call(
        _mlp_body,
        grid_spec=grid_spec,
        out_shape=jax.ShapeDtypeStruct((P_MAX, C_DIM), jnp.float32),
    )(te, x_sorted, w_fc, b_fc, w_proj, b_proj)


def _combine(out_sorted, i0r, i1r, wb0, wb1):
    mesh = plsc.VectorSubcoreMesh(core_axis_name="core",
                                  subcore_axis_name="subcore")

    @functools.partial(
        pl.kernel,
        out_type=jax.ShapeDtypeStruct((N_TOK, C_DIM), jnp.float32),
        mesh=mesh,
        scratch_types=[pltpu.VMEM((TPW // CW, CW), jnp.int32),
                       pltpu.VMEM((TPW // CW, CW), jnp.int32),
                       pltpu.VMEM((CW, LANES), jnp.float32),
                       pltpu.VMEM((CW, LANES), jnp.float32),
                       pltpu.VMEM((CW, C_DIM), jnp.float32),
                       pltpu.VMEM((CW, C_DIM), jnp.float32),
                       pltpu.VMEM((CW, C_DIM), jnp.float32),
                       pltpu.SemaphoreType.DMA,
                       pltpu.SemaphoreType.DMA])
    def k(os_hbm, i0_hbm, i1_hbm, w0_hbm, w1_hbm, y_hbm,
          idx0_v, idx1_v, wb0_v, wb1_v, ra, rb, ybuf, s0, s1):
        w = lax.axis_index("core") * 16 + lax.axis_index("subcore")
        pltpu.sync_copy(i0_hbm.at[w], idx0_v)
        pltpu.sync_copy(i1_hbm.at[w], idx1_v)

        @pl.loop(0, TPW // CW)
        def _(cc):
            c0 = pltpu.async_copy(os_hbm.at[idx0_v.at[cc]], ra, s0)
            c1 = pltpu.async_copy(os_hbm.at[idx1_v.at[cc]], rb, s1)
            pltpu.sync_copy(w0_hbm.at[pl.ds(w * TPW + cc * CW, CW)], wb0_v)
            pltpu.sync_copy(w1_hbm.at[pl.ds(w * TPW + cc * CW, CW)], wb1_v)
            c0.wait()
            c1.wait()

            @pl.loop(0, CW)
            def _(i):
                wa = wb0_v[i, :]
                wb = wb1_v[i, :]
                for ch in range(C_DIM // LANES):
                    sl = pl.ds(ch * LANES, LANES)
                    ybuf[i, sl] = wa * ra[i, sl] + wb * rb[i, sl]

            pltpu.sync_copy(ybuf, y_hbm.at[pl.ds(w * TPW + cc * CW, CW)])

    return k(out_sorted, i0r.reshape(NW, TPW // CW, CW),
             i1r.reshape(NW, TPW // CW, CW), wb0, wb1)


def kernel(x, gate_w, w_fc, b_fc, w_proj, b_proj):
    bx, tx, c = x.shape
    x_flat = x.reshape(-1, c)
    dst, te, wb0, wb1 = _routing(x_flat, gate_w)
    dstr = dst.reshape(TOPK, NW, TPW)
    i0 = dstr[0]
    i1 = dstr[1]
    te_arr = te[0, :M_MAX]
    x_sorted = _dispatch(_cast(x_flat), i0, i1)
    out_sorted = _mlp(
        te_arr, x_sorted,
        w_fc.astype(jnp.bfloat16), b_fc.reshape(N_EXP, 1, H_DIM),
        w_proj.astype(jnp.bfloat16), b_proj.reshape(N_EXP, 1, C_DIM))
    y = _combine(out_sorted, i0, i1, wb0, wb1)
    return y.reshape(bx, tx, c), jnp.asarray(0.0, x.dtype)


# speedup vs baseline: 1.6469x; 1.0113x over previous
import functools

import jax
import jax.numpy as jnp
from jax import lax
from jax.experimental import pallas as pl
from jax.experimental.pallas import tpu as pltpu
from jax.experimental.pallas import tpu_sc as plsc

N_TOK = 4096
C_DIM = 1024
H_DIM = 4096
N_EXP = 8
TOPK = 2
NK = N_TOK * TOPK
TILE = 256
M_MAX = NK // TILE + N_EXP
P_MAX = M_MAX * TILE
LANES = 16
DW = 32
CW = 32


def _routing_body(x_ref, gw_ref, dst_ref, te_ref, wb0_ref, wb1_ref):
    x = x_ref[...]
    gw = gw_ref[...]
    s = jnp.dot(x, gw, preferred_element_type=jnp.float32)
    ids = lax.broadcasted_iota(jnp.int32, s.shape, 1)
    m1 = jnp.max(s, axis=1, keepdims=True)
    i1 = jnp.min(jnp.where(s == m1, ids, N_EXP), axis=1, keepdims=True)
    sm = jnp.where(ids == i1, -jnp.inf, s)
    m2 = jnp.max(sm, axis=1, keepdims=True)
    i2 = jnp.min(jnp.where(sm == m2, ids, N_EXP), axis=1, keepdims=True)
    e2 = jnp.exp(m2 - m1)
    w1 = 1.0 / (1.0 + e2)
    w2 = e2 / (1.0 + e2)

    e_all = jnp.concatenate([i1, i2], axis=0)
    oh = (e_all == lax.broadcasted_iota(jnp.int32, (NK, N_EXP), 1))
    oh = oh.astype(jnp.int32)
    c = oh
    sh = 1
    while sh < NK:
        c = c + jnp.concatenate(
            [jnp.zeros((sh, N_EXP), jnp.int32), c[:-sh, :]], axis=0)
        sh *= 2
    counts = c[NK - 1:NK, :]
    pc = ((counts + TILE - 1) // TILE) * TILE
    t = pc
    for lsh in (1, 2, 4):
        t = t + jnp.concatenate(
            [jnp.zeros((1, lsh), jnp.int32), t[:, :-lsh]], axis=1)
    pad_excl = t - pc
    rank = jnp.sum(c * oh, axis=1, keepdims=True) - 1
    base = jnp.sum(pad_excl * oh, axis=1, keepdims=True)
    dst_ref[...] = base + rank
    mt = lax.broadcasted_iota(jnp.int32, (1, 128), 1) * TILE
    te = jnp.zeros((1, 128), jnp.int32)
    for e in range(1, N_EXP):
        te = te + (pad_excl[:, e:e + 1] <= mt).astype(jnp.int32)
    te_ref[...] = te
    wb0_ref[...] = jnp.broadcast_to(w1, (N_TOK, LANES))
    wb1_ref[...] = jnp.broadcast_to(w2, (N_TOK, LANES))


def _routing(x_flat, gate_w):
    return pl.pallas_call(
        _routing_body,
        out_shape=[
            jax.ShapeDtypeStruct((NK, 1), jnp.int32),
            jax.ShapeDtypeStruct((1, 128), jnp.int32),
            jax.ShapeDtypeStruct((N_TOK, LANES), jnp.float32),
            jax.ShapeDtypeStruct((N_TOK, LANES), jnp.float32),
        ],
    )(x_flat, gate_w)


NW = 32
TPW = N_TOK // NW


def _rtne_bf16_bits(xu):
    return (xu + 0x7FFF + ((xu >> 16) & 1)) >> 16


def _cast_body(x_ref, o_ref):
    x = x_ref[...]
    xu = lax.bitcast_convert_type(x, jnp.uint32)
    lo = _rtne_bf16_bits(xu[:, :C_DIM // 2])
    hi = _rtne_bf16_bits(xu[:, C_DIM // 2:])
    o_ref[...] = lax.bitcast_convert_type(lo | (hi << 16), jnp.int32)


def _cast(x_flat):
    return pl.pallas_call(
        _cast_body,
        grid=(8,),
        in_specs=[pl.BlockSpec((N_TOK // 8, C_DIM), lambda i: (i, 0))],
        out_specs=pl.BlockSpec((N_TOK // 8, C_DIM // 2), lambda i: (i, 0)),
        out_shape=jax.ShapeDtypeStruct((N_TOK, C_DIM // 2), jnp.int32),
    )(x_flat)


def _dispatch(x_i32, i0r, i1r):
    mesh = plsc.VectorSubcoreMesh(core_axis_name="core",
                                  subcore_axis_name="subcore")

    @functools.partial(
        pl.kernel,
        out_type=jax.ShapeDtypeStruct((P_MAX, C_DIM // 2), jnp.int32),
        mesh=mesh,
        scratch_types=[pltpu.VMEM((TPW,), jnp.int32),
                       pltpu.VMEM((TPW,), jnp.int32),
                       pltpu.VMEM((TPW, C_DIM // 2), jnp.int32),
                       pltpu.SemaphoreType.DMA,
                       pltpu.SemaphoreType.DMA])
    def k(x_hbm, i0_hbm, i1_hbm, xs_hbm, idx0_v, idx1_v, xbuf, s0, s1):
        w = lax.axis_index("core") * 16 + lax.axis_index("subcore")
        pltpu.sync_copy(i0_hbm.at[w], idx0_v)
        pltpu.sync_copy(i1_hbm.at[w], idx1_v)
        pltpu.sync_copy(x_hbm.at[pl.ds(w * TPW, TPW)], xbuf)
        c0 = pltpu.async_copy(xbuf, xs_hbm.at[idx0_v], s0)
        c1 = pltpu.async_copy(xbuf, xs_hbm.at[idx1_v], s1)
        c0.wait()
        c1.wait()

    return k(x_i32, i0r, i1r)


def _mlp_body(te_ref, x_ref, wfc_ref, bfc_ref, wpj_ref, bpj_ref, o_ref):
    del te_ref
    wu = lax.bitcast_convert_type(x_ref[...], jnp.uint32)
    xlo = lax.bitcast_convert_type(wu << 16, jnp.float32).astype(jnp.bfloat16)
    xhi = lax.bitcast_convert_type(wu & jnp.uint32(0xFFFF0000),
                                   jnp.float32).astype(jnp.bfloat16)
    dn = (((1,), (0,)), ((), ()))
    acc = bpj_ref[0]
    HC = 1024
    for hc in range(H_DIM // HC):
        sl = pl.ds(hc * HC, HC)
        h = (lax.dot_general(xlo, wfc_ref[0, :C_DIM // 2, sl], dn,
                             preferred_element_type=jnp.float32)
             + lax.dot_general(xhi, wfc_ref[0, C_DIM // 2:, sl], dn,
                               preferred_element_type=jnp.float32))
        h = h + bfc_ref[0, :, sl]
        h = 0.5 * h * (1.0 + lax.erf(h * 0.7071067811865476))
        acc = acc + lax.dot_general(h.astype(jnp.bfloat16), wpj_ref[0, sl], dn,
                                    preferred_element_type=jnp.float32)
    o_ref[...] = acc


def _mlp(te, x_sorted, w_fc, b_fc, w_proj, b_proj):
    grid_spec = pltpu.PrefetchScalarGridSpec(
        num_scalar_prefetch=1,
        grid=(M_MAX,),
        in_specs=[
            pl.BlockSpec((TILE, C_DIM // 2), lambda m, te: (m, 0)),
            pl.BlockSpec((1, C_DIM, H_DIM), lambda m, te: (te[m], 0, 0)),
            pl.BlockSpec((1, 1, H_DIM), lambda m, te: (te[m], 0, 0)),
            pl.BlockSpec((1, H_DIM, C_DIM), lambda m, te: (te[m], 0, 0)),
            pl.BlockSpec((1, 1, C_DIM), lambda m, te: (te[m], 0, 0)),
        ],
        out_specs=pl.BlockSpec((TILE, C_DIM), lambda m, te: (m, 0)),
    )
    return pl.pallas_call(
        _mlp_body,
        grid_spec=grid_spec,
        out_shape=jax.ShapeDtypeStruct((P_MAX, C_DIM), jnp.float32),
    )(te, x_sorted, w_fc, b_fc, w_proj, b_proj)


def _combine(out_sorted, i0r, i1r, wb0, wb1):
    mesh = plsc.VectorSubcoreMesh(core_axis_name="core",
                                  subcore_axis_name="subcore")

    @functools.partial(
        pl.kernel,
        out_type=jax.ShapeDtypeStruct((N_TOK, C_DIM), jnp.float32),
        mesh=mesh,
        scratch_types=[pltpu.VMEM((TPW // CW, CW), jnp.int32),
                       pltpu.VMEM((TPW // CW, CW), jnp.int32),
                       pltpu.VMEM((CW, LANES), jnp.float32),
                       pltpu.VMEM((CW, LANES), jnp.float32),
                       pltpu.VMEM((CW, C_DIM), jnp.float32),
                       pltpu.VMEM((CW, C_DIM), jnp.float32),
                       pltpu.VMEM((CW, C_DIM), jnp.float32),
                       pltpu.SemaphoreType.DMA,
                       pltpu.SemaphoreType.DMA])
    def k(os_hbm, i0_hbm, i1_hbm, w0_hbm, w1_hbm, y_hbm,
          idx0_v, idx1_v, wb0_v, wb1_v, ra, rb, ybuf, s0, s1):
        w = lax.axis_index("core") * 16 + lax.axis_index("subcore")
        pltpu.sync_copy(i0_hbm.at[w], idx0_v)
        pltpu.sync_copy(i1_hbm.at[w], idx1_v)

        @pl.loop(0, TPW // CW)
        def _(cc):
            c0 = pltpu.async_copy(os_hbm.at[idx0_v.at[cc]], ra, s0)
            c1 = pltpu.async_copy(os_hbm.at[idx1_v.at[cc]], rb, s1)
            pltpu.sync_copy(w0_hbm.at[pl.ds(w * TPW + cc * CW, CW)], wb0_v)
            pltpu.sync_copy(w1_hbm.at[pl.ds(w * TPW + cc * CW, CW)], wb1_v)
            c0.wait()
            c1.wait()

            @pl.loop(0, CW)
            def _(i):
                wa = wb0_v[i, :]
                wb = wb1_v[i, :]
                for ch in range(C_DIM // LANES):
                    sl = pl.ds(ch * LANES, LANES)
                    ybuf[i, sl] = wa * ra[i, sl] + wb * rb[i, sl]

            pltpu.sync_copy(ybuf, y_hbm.at[pl.ds(w * TPW + cc * CW, CW)])

    return k(out_sorted, i0r.reshape(NW, TPW // CW, CW),
             i1r.reshape(NW, TPW // CW, CW), wb0, wb1)


def kernel(x, gate_w, w_fc, b_fc, w_proj, b_proj):
    bx, tx, c = x.shape
    x_flat = x.reshape(-1, c)
    dst, te, wb0, wb1 = _routing(x_flat, gate_w)
    dstr = dst.reshape(TOPK, NW, TPW)
    i0 = dstr[0]
    i1 = dstr[1]
    te_arr = te[0, :M_MAX]
    x_sorted = _dispatch(_cast(x_flat), i0, i1)
    out_sorted = _mlp(
        te_arr, x_sorted,
        w_fc.astype(jnp.bfloat16), b_fc.reshape(N_EXP, 1, H_DIM),
        w_proj.astype(jnp.bfloat16), b_proj.reshape(N_EXP, 1, C_DIM))
    y = _combine(out_sorted, i0, i1, wb0, wb1)
    return y.reshape(bx, tx, c), jnp.asarray(0.0, x.dtype)
